# Initial kernel scaffold; baseline (speedup 1.0000x reference)
import functools

import jax
import jax.numpy as jnp
from jax import lax
from jax.experimental import pallas as pl
from jax.experimental.pallas import tpu as pltpu
from jax.experimental.pallas import tpu_sc as plsc

N = 10000
F = 128
NC = 2
NS = 16
NW = NC * NS
CHUNK = 128
NCHUNK = 79
EPT = NCHUNK * CHUNK
E_PAD = NW * EPT
NROW = 10016
RPT = NROW // NS
OPT = N // NS

_mesh = plsc.VectorSubcoreMesh(core_axis_name="c", subcore_axis_name="s")


@functools.partial(
    pl.kernel,
    out_type=jax.ShapeDtypeStruct((NC, NROW), jnp.float32),
    mesh=_mesh,
    scratch_types=[
        pltpu.VMEM((CHUNK,), jnp.int32),
        pltpu.VMEM((CHUNK,), jnp.float32),
        pltpu.VMEM_SHARED((NROW,), jnp.float32),
        pltpu.SemaphoreType.DMA,
    ],
)
def _deg_kernel(dst_hbm, zeros_hbm, ones_hbm, out_hbm, didx, ones_v, acc, sem):
    cid = lax.axis_index("c")
    tid = lax.axis_index("s")
    wid = tid * NC + cid

    @pl.when(tid == 0)
    def _zero():
        pltpu.sync_copy(zeros_hbm, acc)

    pltpu.sync_copy(ones_hbm, ones_v)
    plsc.subcore_barrier()

    def body(i, carry):
        base = wid * EPT + i * CHUNK
        pltpu.sync_copy(dst_hbm.at[pl.ds(base, CHUNK)], didx)
        pltpu.sync_copy(ones_v, acc.at[didx], add=True)
        return carry

    lax.fori_loop(0, NCHUNK, body, 0)
    plsc.subcore_barrier()

    @pl.when(tid == 0)
    def _out():
        pltpu.sync_copy(acc, out_hbm.at[cid])


@functools.partial(
    pl.kernel,
    out_type=jax.ShapeDtypeStruct((NC, N, F), jnp.float32),
    mesh=_mesh,
    scratch_types=[
        pltpu.VMEM((CHUNK,), jnp.int32),
        pltpu.VMEM((CHUNK,), jnp.int32),
        pltpu.VMEM((CHUNK, F), jnp.float32),
        pltpu.VMEM_SHARED((NROW, F), jnp.float32),
        pltpu.SemaphoreType.DMA,
    ],
)
def _edge_kernel(g_hbm, src_hbm, dst_hbm, zrows_hbm, out_hbm,
                 sidx, didx, rows, acc, sem):
    cid = lax.axis_index("c")
    tid = lax.axis_index("s")
    wid = tid * NC + cid

    pltpu.sync_copy(zrows_hbm.at[pl.ds(tid * RPT, RPT)],
                    acc.at[pl.ds(tid * RPT, RPT)])
    plsc.subcore_barrier()

    def body(i, carry):
        base = wid * EPT + i * CHUNK
        pltpu.sync_copy(src_hbm.at[pl.ds(base, CHUNK)], sidx)
        pltpu.sync_copy(dst_hbm.at[pl.ds(base, CHUNK)], didx)
        pltpu.async_copy(g_hbm.at[sidx], rows, sem).wait()
        pltpu.sync_copy(rows, acc.at[didx], add=True)
        return carry

    lax.fori_loop(0, NCHUNK, body, 0)
    plsc.subcore_barrier()

    pltpu.sync_copy(acc.at[pl.ds(tid * OPT, OPT)],
                    out_hbm.at[cid, pl.ds(tid * OPT, OPT)])


_R = 1000


def _stage1_body(x_ref, w_ref, d0_ref, d1_ref, g_ref, dinv_ref):
    deg = d0_ref[...] + d1_ref[...] + 1.0
    dv = lax.rsqrt(deg)
    dinv_ref[...] = dv
    g_ref[...] = dv * jnp.dot(x_ref[...], w_ref[...],
                              preferred_element_type=jnp.float32)


def _stage2_body(p0_ref, p1_ref, g_ref, dinv_ref, b_ref, w_ref,
                 h_ref, g2_ref):
    dv = dinv_ref[...]
    h = jnp.maximum(dv * (p0_ref[...] + p1_ref[...] + g_ref[...]) + b_ref[...],
                    0.0)
    h_ref[...] = h
    g2_ref[...] = dv * jnp.dot(h, w_ref[...], preferred_element_type=jnp.float32)


def _stage3_body(p0_ref, p1_ref, g_ref, dinv_ref, b_ref, h1_ref,
                 lw_ref, lb_ref, o_ref):
    dv = dinv_ref[...]
    h2 = jnp.maximum(dv * (p0_ref[...] + p1_ref[...] + g_ref[...]) + b_ref[...],
                     0.0)
    hm = jnp.maximum(h1_ref[...], h2)
    z = jnp.dot(hm, lw_ref[...], preferred_element_type=jnp.float32) + lb_ref[...]
    m = jnp.max(z, axis=1, keepdims=True)
    lse = jnp.log(jnp.sum(jnp.exp(z - m), axis=1, keepdims=True))
    o_ref[...] = z - m - lse


def _row_spec(cols):
    return pl.BlockSpec((_R, cols), lambda i: (i, 0))


def _full_spec(r, c):
    return pl.BlockSpec((r, c), lambda i: (0, 0))


def kernel(x, edge_index, W1, b1, W2, b2, lin_W, lin_b):
    src = edge_index[0].astype(jnp.int32)
    dst = edge_index[1].astype(jnp.int32)
    e = src.shape[0]
    pad = E_PAD - e
    src = jnp.concatenate([src, jnp.zeros((pad,), jnp.int32)])
    dst = jnp.concatenate([dst, jnp.full((pad,), N, jnp.int32)])

    zeros_vec = jnp.zeros((NROW,), jnp.float32)
    ones_chunk = jnp.ones((CHUNK,), jnp.float32)
    zrows = jnp.zeros((NROW, F), jnp.float32)

    degp = _deg_kernel(dst, zeros_vec, ones_chunk)
    deg0 = degp[0, :N].reshape(N, 1)
    deg1 = degp[1, :N].reshape(N, 1)

    grid = N // _R
    g1, dinv = pl.pallas_call(
        _stage1_body,
        grid=(grid,),
        in_specs=[_row_spec(F), _full_spec(F, F), _row_spec(1), _row_spec(1)],
        out_specs=[_row_spec(F), _row_spec(1)],
        out_shape=[jax.ShapeDtypeStruct((N, F), jnp.float32),
                   jax.ShapeDtypeStruct((N, 1), jnp.float32)],
    )(x, W1, deg0, deg1)

    p1 = _edge_kernel(g1, src, dst, zrows)

    h1, g2 = pl.pallas_call(
        _stage2_body,
        grid=(grid,),
        in_specs=[_row_spec(F), _row_spec(F), _row_spec(F), _row_spec(1),
                  _full_spec(1, F), _full_spec(F, F)],
        out_specs=[_row_spec(F), _row_spec(F)],
        out_shape=[jax.ShapeDtypeStruct((N, F), jnp.float32),
                   jax.ShapeDtypeStruct((N, F), jnp.float32)],
    )(p1[0], p1[1], g1, dinv, b1.reshape(1, F), W2)

    p2 = _edge_kernel(g2, src, dst, zrows)

    C = lin_W.shape[1]
    out = pl.pallas_call(
        _stage3_body,
        grid=(grid,),
        in_specs=[_row_spec(F), _row_spec(F), _row_spec(F), _row_spec(1),
                  _full_spec(1, F), _row_spec(F), _full_spec(F, C),
                  _full_spec(1, C)],
        out_specs=pl.BlockSpec((_R, C), lambda i: (i, 0)),
        out_shape=jax.ShapeDtypeStruct((N, C), jnp.float32),
    )(p2[0], p2[1], g2, dinv, b2.reshape(1, F), h1, lin_W,
      lin_b.reshape(1, C))

    return out


# speedup vs baseline: 9.9868x; 9.9868x over previous
import functools

import jax
import jax.numpy as jnp
from jax import lax
from jax.experimental import pallas as pl
from jax.experimental.pallas import tpu as pltpu
from jax.experimental.pallas import tpu_sc as plsc

N = 10000
F = 128
NC = 2
NS = 16
NW = NC * NS
CHUNK = 128
NCHUNK = 79
EPT = NCHUNK * CHUNK
E_PAD = NW * EPT
NROW = 10112
RPT = NROW // NS

_mesh = plsc.VectorSubcoreMesh(core_axis_name="c", subcore_axis_name="s")


@functools.partial(
    pl.kernel,
    out_type=jax.ShapeDtypeStruct((NC, NROW), jnp.float32),
    mesh=_mesh,
    scratch_types=[
        pltpu.VMEM((CHUNK,), jnp.int32),
        pltpu.VMEM((CHUNK,), jnp.float32),
        pltpu.VMEM_SHARED((NROW,), jnp.float32),
        pltpu.SemaphoreType.DMA,
    ],
)
def _deg_kernel(dst_hbm, zeros_hbm, ones_hbm, out_hbm, didx, ones_v, acc, sem):
    cid = lax.axis_index("c")
    tid = lax.axis_index("s")
    wid = tid * NC + cid

    @pl.when(tid == 0)
    def _zero():
        pltpu.sync_copy(zeros_hbm, acc)

    pltpu.sync_copy(ones_hbm, ones_v)
    plsc.subcore_barrier()

    def body(i, carry):
        base = wid * EPT + i * CHUNK
        pltpu.sync_copy(dst_hbm.at[pl.ds(base, CHUNK)], didx)
        pltpu.sync_copy(ones_v, acc.at[didx], add=True)
        return carry

    lax.fori_loop(0, NCHUNK, body, 0)
    plsc.subcore_barrier()

    @pl.when(tid == 0)
    def _out():
        pltpu.sync_copy(acc, out_hbm.at[cid])


@functools.partial(
    pl.kernel,
    out_type=jax.ShapeDtypeStruct((NC, NROW, F), jnp.float32),
    mesh=_mesh,
    scratch_types=[
        pltpu.VMEM((CHUNK,), jnp.int32),
        pltpu.VMEM((CHUNK,), jnp.int32),
        pltpu.VMEM((CHUNK, F), jnp.float32),
        pltpu.VMEM_SHARED((NROW, F), jnp.float32),
        pltpu.SemaphoreType.DMA,
    ],
)
def _edge_kernel(g_hbm, src_hbm, dst_hbm, zrows_hbm, out_hbm,
                 sidx, didx, rows, acc, sem):
    cid = lax.axis_index("c")
    tid = lax.axis_index("s")
    wid = tid * NC + cid

    pltpu.sync_copy(zrows_hbm.at[pl.ds(tid * RPT, RPT)],
                    acc.at[pl.ds(tid * RPT, RPT)])
    plsc.subcore_barrier()

    def body(i, carry):
        base = wid * EPT + i * CHUNK
        pltpu.sync_copy(src_hbm.at[pl.ds(base, CHUNK)], sidx)
        pltpu.sync_copy(dst_hbm.at[pl.ds(base, CHUNK)], didx)
        pltpu.async_copy(g_hbm.at[sidx], rows, sem).wait()
        pltpu.sync_copy(rows, acc.at[didx], add=True)
        return carry

    lax.fori_loop(0, NCHUNK, body, 0)
    plsc.subcore_barrier()

    pltpu.sync_copy(acc.at[pl.ds(tid * RPT, RPT)],
                    out_hbm.at[cid, pl.ds(tid * RPT, RPT)])


_R = 1000


def _stage1_body(x_ref, w_ref, d0_ref, d1_ref, g_ref, dinv_ref):
    deg = d0_ref[...] + d1_ref[...] + 1.0
    dv = lax.rsqrt(deg)
    dinv_ref[...] = dv
    g_ref[...] = dv * jnp.dot(x_ref[...], w_ref[...],
                              preferred_element_type=jnp.float32)


def _stage2_body(p0_ref, p1_ref, g_ref, dinv_ref, b_ref, w_ref,
                 h_ref, g2_ref):
    dv = dinv_ref[...]
    h = jnp.maximum(dv * (p0_ref[...] + p1_ref[...] + g_ref[...]) + b_ref[...],
                    0.0)
    h_ref[...] = h
    g2_ref[...] = dv * jnp.dot(h, w_ref[...], preferred_element_type=jnp.float32)


def _stage3_body(p0_ref, p1_ref, g_ref, dinv_ref, b_ref, h1_ref,
                 lw_ref, lb_ref, o_ref):
    dv = dinv_ref[...]
    h2 = jnp.maximum(dv * (p0_ref[...] + p1_ref[...] + g_ref[...]) + b_ref[...],
                     0.0)
    hm = jnp.maximum(h1_ref[...], h2)
    z = jnp.dot(hm, lw_ref[...], preferred_element_type=jnp.float32) + lb_ref[...]
    m = jnp.max(z, axis=1, keepdims=True)
    lse = jnp.log(jnp.sum(jnp.exp(z - m), axis=1, keepdims=True))
    o_ref[...] = z - m - lse


def _row_spec(cols):
    return pl.BlockSpec((_R, cols), lambda i: (i, 0))


def _full_spec(r, c):
    return pl.BlockSpec((r, c), lambda i: (0, 0))


def kernel(x, edge_index, W1, b1, W2, b2, lin_W, lin_b):
    src = edge_index[0].astype(jnp.int32)
    dst = edge_index[1].astype(jnp.int32)
    e = src.shape[0]
    pad = E_PAD - e
    src = jnp.concatenate([src, jnp.zeros((pad,), jnp.int32)])
    dst = jnp.concatenate([dst, jnp.full((pad,), N, jnp.int32)])

    zeros_vec = jnp.zeros((NROW,), jnp.float32)
    ones_chunk = jnp.ones((CHUNK,), jnp.float32)
    zrows = jnp.zeros((NROW, F), jnp.float32)

    degp = _deg_kernel(dst, zeros_vec, ones_chunk)
    deg0 = degp[0, :N].reshape(N, 1)
    deg1 = degp[1, :N].reshape(N, 1)

    grid = N // _R
    g1, dinv = pl.pallas_call(
        _stage1_body,
        grid=(grid,),
        in_specs=[_row_spec(F), _full_spec(F, F), _row_spec(1), _row_spec(1)],
        out_specs=[_row_spec(F), _row_spec(1)],
        out_shape=[jax.ShapeDtypeStruct((N, F), jnp.float32),
                   jax.ShapeDtypeStruct((N, 1), jnp.float32)],
    )(x, W1, deg0, deg1)

    p1 = _edge_kernel(g1, src, dst, zrows)
    p1 = p1[:, :N]

    h1, g2 = pl.pallas_call(
        _stage2_body,
        grid=(grid,),
        in_specs=[_row_spec(F), _row_spec(F), _row_spec(F), _row_spec(1),
                  _full_spec(1, F), _full_spec(F, F)],
        out_specs=[_row_spec(F), _row_spec(F)],
        out_shape=[jax.ShapeDtypeStruct((N, F), jnp.float32),
                   jax.ShapeDtypeStruct((N, F), jnp.float32)],
    )(p1[0], p1[1], g1, dinv, b1.reshape(1, F), W2)

    p2 = _edge_kernel(g2, src, dst, zrows)
    p2 = p2[:, :N]

    C = lin_W.shape[1]
    out = pl.pallas_call(
        _stage3_body,
        grid=(grid,),
        in_specs=[_row_spec(F), _row_spec(F), _row_spec(F), _row_spec(1),
                  _full_spec(1, F), _row_spec(F), _full_spec(F, C),
                  _full_spec(1, C)],
        out_specs=pl.BlockSpec((_R, C), lambda i: (i, 0)),
        out_shape=jax.ShapeDtypeStruct((N, C), jnp.float32),
    )(p2[0], p2[1], g2, dinv, b2.reshape(1, F), h1, lin_W,
      lin_b.reshape(1, C))

    return out


# speedup vs baseline: 10.4609x; 1.0475x over previous
import functools

import jax
import jax.numpy as jnp
from jax import lax
from jax.experimental import pallas as pl
from jax.experimental.pallas import tpu as pltpu
from jax.experimental.pallas import tpu_sc as plsc

N = 10000
F = 128
NC = 2
NS = 16
NW = NC * NS
CHUNK = 128
NCHUNK = 80
HC = 40
EPT = NCHUNK * CHUNK
E_PAD = NW * EPT
NROW = 10112
RPT = NROW // NS

_mesh = plsc.VectorSubcoreMesh(core_axis_name="c", subcore_axis_name="s")


@functools.partial(
    pl.kernel,
    out_type=jax.ShapeDtypeStruct((NC, NROW), jnp.float32),
    mesh=_mesh,
    scratch_types=[
        pltpu.VMEM((NCHUNK, CHUNK), jnp.int32),
        pltpu.VMEM((CHUNK,), jnp.float32),
        pltpu.VMEM_SHARED((NROW,), jnp.float32),
        pltpu.SemaphoreType.DMA,
    ],
)
def _deg_kernel(dst_hbm, zeros_hbm, ones_hbm, out_hbm, didx, ones_v, acc, sem):
    cid = lax.axis_index("c")
    tid = lax.axis_index("s")
    wid = tid * NC + cid

    @pl.when(tid == 0)
    def _zero():
        pltpu.sync_copy(zeros_hbm, acc)

    pltpu.sync_copy(ones_hbm, ones_v)
    pltpu.sync_copy(dst_hbm.at[wid], didx)
    plsc.subcore_barrier()

    def body(i, carry):
        pltpu.sync_copy(ones_v, acc.at[didx.at[i]], add=True)
        return carry

    lax.fori_loop(0, NCHUNK, body, 0)
    plsc.subcore_barrier()

    @pl.when(tid == 0)
    def _out():
        pltpu.sync_copy(acc, out_hbm.at[cid])


@functools.partial(
    pl.kernel,
    out_type=jax.ShapeDtypeStruct((NC, NROW, F), jnp.float32),
    mesh=_mesh,
    scratch_types=[
        pltpu.VMEM((HC, CHUNK), jnp.int32),
        pltpu.VMEM((HC, CHUNK), jnp.int32),
        pltpu.VMEM((CHUNK, F), jnp.float32),
        pltpu.VMEM((CHUNK, F), jnp.float32),
        pltpu.VMEM_SHARED((NROW, F), jnp.float32),
        pltpu.SemaphoreType.DMA,
    ],
)
def _edge_kernel(g_hbm, src_hbm, dst_hbm, zrows_hbm, out_hbm,
                 sidx, didx, rows_a, rows_b, acc, sem):
    cid = lax.axis_index("c")
    tid = lax.axis_index("s")
    wid = tid * NC + cid

    pltpu.sync_copy(zrows_hbm.at[pl.ds(tid * RPT, RPT)],
                    acc.at[pl.ds(tid * RPT, RPT)])
    plsc.subcore_barrier()

    for h in range(NCHUNK // HC):
        pltpu.sync_copy(src_hbm.at[wid, pl.ds(h * HC, HC)], sidx)
        pltpu.sync_copy(dst_hbm.at[wid, pl.ds(h * HC, HC)], didx)
        pltpu.async_copy(g_hbm.at[sidx.at[0]], rows_a, sem)

        def body(j, carry):
            i = j * 2
            pltpu.async_copy(g_hbm.at[sidx.at[i + 1]], rows_b, sem)
            pltpu.make_async_copy(g_hbm.at[sidx.at[i]], rows_a, sem).wait()
            pltpu.sync_copy(rows_a, acc.at[didx.at[i]], add=True)

            @pl.when(j < HC // 2 - 1)
            def _prefetch():
                pltpu.async_copy(g_hbm.at[sidx.at[i + 2]], rows_a, sem)

            pltpu.make_async_copy(g_hbm.at[sidx.at[i + 1]], rows_b, sem).wait()
            pltpu.sync_copy(rows_b, acc.at[didx.at[i + 1]], add=True)
            return carry

        lax.fori_loop(0, HC // 2, body, 0)
    plsc.subcore_barrier()

    pltpu.sync_copy(acc.at[pl.ds(tid * RPT, RPT)],
                    out_hbm.at[cid, pl.ds(tid * RPT, RPT)])


_R = 1000


def _stage1_body(x_ref, w_ref, d0_ref, d1_ref, g_ref, dinv_ref):
    deg = d0_ref[...] + d1_ref[...] + 1.0
    dv = lax.rsqrt(deg)
    dinv_ref[...] = dv
    g_ref[...] = dv * jnp.dot(x_ref[...], w_ref[...],
                              preferred_element_type=jnp.float32)


def _stage2_body(p0_ref, p1_ref, g_ref, dinv_ref, b_ref, w_ref,
                 h_ref, g2_ref):
    dv = dinv_ref[...]
    h = jnp.maximum(dv * (p0_ref[...] + p1_ref[...] + g_ref[...]) + b_ref[...],
                    0.0)
    h_ref[...] = h
    g2_ref[...] = dv * jnp.dot(h, w_ref[...], preferred_element_type=jnp.float32)


def _stage3_body(p0_ref, p1_ref, g_ref, dinv_ref, b_ref, h1_ref,
                 lw_ref, lb_ref, o_ref):
    dv = dinv_ref[...]
    h2 = jnp.maximum(dv * (p0_ref[...] + p1_ref[...] + g_ref[...]) + b_ref[...],
                     0.0)
    hm = jnp.maximum(h1_ref[...], h2)
    z = jnp.dot(hm, lw_ref[...], preferred_element_type=jnp.float32) + lb_ref[...]
    m = jnp.max(z, axis=1, keepdims=True)
    lse = jnp.log(jnp.sum(jnp.exp(z - m), axis=1, keepdims=True))
    o_ref[...] = z - m - lse


def _row_spec(cols):
    return pl.BlockSpec((_R, cols), lambda i: (i, 0))


def _full_spec(r, c):
    return pl.BlockSpec((r, c), lambda i: (0, 0))


def kernel(x, edge_index, W1, b1, W2, b2, lin_W, lin_b):
    src = edge_index[0].astype(jnp.int32)
    dst = edge_index[1].astype(jnp.int32)
    e = src.shape[0]
    pad = E_PAD - e
    src = jnp.concatenate([src, jnp.zeros((pad,), jnp.int32)])
    dst = jnp.concatenate([dst, jnp.full((pad,), N, jnp.int32)])
    src = src.reshape(NW, NCHUNK, CHUNK)
    dst = dst.reshape(NW, NCHUNK, CHUNK)

    zeros_vec = jnp.zeros((NROW,), jnp.float32)
    ones_chunk = jnp.ones((CHUNK,), jnp.float32)
    zrows = jnp.zeros((NROW, F), jnp.float32)

    degp = _deg_kernel(dst, zeros_vec, ones_chunk)
    deg0 = degp[0, :N].reshape(N, 1)
    deg1 = degp[1, :N].reshape(N, 1)

    grid = N // _R
    g1, dinv = pl.pallas_call(
        _stage1_body,
        grid=(grid,),
        in_specs=[_row_spec(F), _full_spec(F, F), _row_spec(1), _row_spec(1)],
        out_specs=[_row_spec(F), _row_spec(1)],
        out_shape=[jax.ShapeDtypeStruct((N, F), jnp.float32),
                   jax.ShapeDtypeStruct((N, 1), jnp.float32)],
    )(x, W1, deg0, deg1)

    p1 = _edge_kernel(g1, src, dst, zrows)
    p1 = p1[:, :N]

    h1, g2 = pl.pallas_call(
        _stage2_body,
        grid=(grid,),
        in_specs=[_row_spec(F), _row_spec(F), _row_spec(F), _row_spec(1),
                  _full_spec(1, F), _full_spec(F, F)],
        out_specs=[_row_spec(F), _row_spec(F)],
        out_shape=[jax.ShapeDtypeStruct((N, F), jnp.float32),
                   jax.ShapeDtypeStruct((N, F), jnp.float32)],
    )(p1[0], p1[1], g1, dinv, b1.reshape(1, F), W2)

    p2 = _edge_kernel(g2, src, dst, zrows)
    p2 = p2[:, :N]

    C = lin_W.shape[1]
    out = pl.pallas_call(
        _stage3_body,
        grid=(grid,),
        in_specs=[_row_spec(F), _row_spec(F), _row_spec(F), _row_spec(1),
                  _full_spec(1, F), _row_spec(F), _full_spec(F, C),
                  _full_spec(1, C)],
        out_specs=pl.BlockSpec((_R, C), lambda i: (i, 0)),
        out_shape=jax.ShapeDtypeStruct((N, C), jnp.float32),
    )(p2[0], p2[1], g2, dinv, b2.reshape(1, F), h1, lin_W,
      lin_b.reshape(1, C))

    return out
